# Initial kernel scaffold; baseline (speedup 1.0000x reference)
import jax
import jax.numpy as jnp
from jax.experimental import pallas as pl


def kernel(x, edge_feat, edge_index, lg_edge_index, rel_emb, params):
    raise NotImplementedError("write your pallas kernel here")


# speedup vs baseline: 32.7893x; 32.7893x over previous
import functools
import math

import jax
import jax.numpy as jnp
from jax import lax
from jax.experimental import pallas as pl
from jax.experimental.pallas import tpu as pltpu
from jax.experimental.pallas import tpu_sc as plsc

F32 = jnp.float32
I32 = jnp.int32

NC = 2
NS = 16
LANES = 16

H = 8
DK = 16
ROWW = 144


def _ln(h, g, b):
    mu = jnp.mean(h, axis=-1, keepdims=True)
    d = h - mu
    var = jnp.mean(d * d, axis=-1, keepdims=True)
    return d * lax.rsqrt(var + 1e-5) * g + b


def _dot(a, b):
    return jnp.dot(a, b, preferred_element_type=F32)


def _lgx_call(feat3, rel_emb, E, R, ED, BF):
    def body(f_ref, t_ref, o_ref):
        f = f_ref[0, 0, :]
        oh = (f[:, None] == lax.broadcasted_iota(I32, (BF, R), 1)).astype(F32)
        o_ref[...] = _dot(oh, t_ref[...])

    return pl.pallas_call(
        body,
        grid=(E // BF,),
        in_specs=[
            pl.BlockSpec((1, 1, BF), lambda i: (i, 0, 0)),
            pl.BlockSpec((R, ED), lambda i: (0, 0)),
        ],
        out_specs=pl.BlockSpec((BF, ED), lambda i: (i, 0)),
        out_shape=jax.ShapeDtypeStruct((E, ED), F32),
    )(feat3, rel_emb)


def _qkv_call(x, W, b, N, D, BN):
    def body(x_ref, w_ref, b_ref, q_ref, k_ref, v_ref):
        o = _dot(x_ref[...], w_ref[...]) + b_ref[...]
        q_ref[...] = o[:, :D]
        k_ref[...] = o[:, D:2 * D]
        v_ref[...] = o[:, 2 * D:]

    outs = [jax.ShapeDtypeStruct((N, D), F32)] * 3
    return pl.pallas_call(
        body,
        grid=(N // BN,),
        in_specs=[
            pl.BlockSpec((BN, D), lambda i: (i, 0)),
            pl.BlockSpec((D, 3 * D), lambda i: (0, 0)),
            pl.BlockSpec((1, 3 * D), lambda i: (0, 0)),
        ],
        out_specs=[pl.BlockSpec((BN, D), lambda i: (i, 0))] * 3,
        out_shape=outs,
    )(x, W, b)


def _score_call(qd, kg, vg, lg, SUM16, SPREAD, M, D, ED, BE, with_lg):
    def body(*refs):
        if with_lg:
            q_ref, k_ref, v_ref, l_ref, s16_ref, sp_ref, o_ref = refs
            lgt = jnp.concatenate([l_ref[...]] * H, axis=1)
            ke = k_ref[...] + lgt
            ve = v_ref[...] + lgt
        else:
            q_ref, k_ref, v_ref, s16_ref, sp_ref, o_ref = refs
            ke = k_ref[...]
            ve = v_ref[...]
        prod = q_ref[...] * ke
        s = _dot(prod, s16_ref[...]) * (1.0 / 4.0)
        sc = jnp.exp(jnp.clip(s, -5.0, 5.0))
        wide = _dot(sc, sp_ref[...])
        wv = wide * ve
        o_ref[...] = jnp.concatenate(
            [wv, sc, jnp.zeros((BE, ROWW - D - H), F32)], axis=1)

    in_specs = [
        pl.BlockSpec((BE, D), lambda i: (i, 0)),
        pl.BlockSpec((BE, D), lambda i: (i, 0)),
        pl.BlockSpec((BE, D), lambda i: (i, 0)),
    ]
    args = [qd, kg, vg]
    if with_lg:
        in_specs.append(pl.BlockSpec((BE, ED), lambda i: (i, 0)))
        args.append(lg)
    in_specs += [
        pl.BlockSpec((D, H), lambda i: (0, 0)),
        pl.BlockSpec((H, D), lambda i: (0, 0)),
    ]
    args += [SUM16, SPREAD]
    return pl.pallas_call(
        body,
        grid=(M // BE,),
        in_specs=in_specs,
        out_specs=pl.BlockSpec((BE, ROWW), lambda i: (i, 0)),
        out_shape=jax.ShapeDtypeStruct((M, ROWW), F32),
    )(*args)


def _edense_call(lg, sxg, dxg, W3, b3, E, D, ED, BE):
    def body(l_ref, s_ref, d_ref, w_ref, b_ref, q_ref, k_ref, v_ref):
        o = _dot(l_ref[...], w_ref[...]) + b_ref[...]
        q_ref[...] = o[:, :D] + d_ref[...]
        k_ref[...] = o[:, D:2 * D] + s_ref[...]
        v_ref[...] = o[:, 2 * D:] + s_ref[...]

    outs = [jax.ShapeDtypeStruct((E, D), F32)] * 3
    return pl.pallas_call(
        body,
        grid=(E // BE,),
        in_specs=[
            pl.BlockSpec((BE, ED), lambda i: (i, 0)),
            pl.BlockSpec((BE, D), lambda i: (i, 0)),
            pl.BlockSpec((BE, D), lambda i: (i, 0)),
            pl.BlockSpec((ED, 3 * D), lambda i: (0, 0)),
            pl.BlockSpec((1, 3 * D), lambda i: (0, 0)),
        ],
        out_specs=[pl.BlockSpec((BE, D), lambda i: (i, 0))] * 3,
        out_shape=outs,
    )(lg, sxg, dxg, W3, b3)


def _npost_call(p0, p1, x, SPREAD, Wo, bo, g1, b1, W1, bb1, W2, bb2, g2, b2,
                N, D, BN):
    def body(p0_ref, p1_ref, x_ref, sp_ref, wo_ref, bo_ref, g1_ref, b1_ref,
             w1_ref, bb1_ref, w2_ref, bb2_ref, g2_ref, b2_ref, o_ref):
        wv = p0_ref[...] + p1_ref[...]
        z = wv[:, D:D + H]
        inv = 1.0 / (z + 1e-9)
        o = wv[:, :D] * _dot(inv, sp_ref[...])
        u = x_ref[...] + _dot(o, wo_ref[...]) + bo_ref[...]
        u = _ln(u, g1_ref[...], b1_ref[...])
        f = u + _dot(jnp.maximum(_dot(u, w1_ref[...]) + bb1_ref[...], 0.0),
                     w2_ref[...]) + bb2_ref[...]
        o_ref[...] = _ln(f, g2_ref[...], b2_ref[...])

    row = lambda i: (i, 0)
    full = lambda i: (0, 0)
    return pl.pallas_call(
        body,
        grid=(N // BN,),
        in_specs=[
            pl.BlockSpec((BN, ROWW), row),
            pl.BlockSpec((BN, ROWW), row),
            pl.BlockSpec((BN, D), row),
            pl.BlockSpec((H, D), full),
            pl.BlockSpec((D, D), full),
            pl.BlockSpec((1, D), full),
            pl.BlockSpec((1, D), full),
            pl.BlockSpec((1, D), full),
            pl.BlockSpec((D, 4 * D), full),
            pl.BlockSpec((1, 4 * D), full),
            pl.BlockSpec((4 * D, D), full),
            pl.BlockSpec((1, D), full),
            pl.BlockSpec((1, D), full),
            pl.BlockSpec((1, D), full),
        ],
        out_specs=pl.BlockSpec((BN, D), row),
        out_shape=jax.ShapeDtypeStruct((N, D), F32),
    )(p0, p1, x, SPREAD, Wo, bo, g1, b1, W1, bb1, W2, bb2, g2, b2)


def _epost_call(wv2, lg, SPREAD, Wo, bo, g1, b1, W1, bb1, W2, bb2, g2, b2,
                E, D, ED, BE):
    def body(wv_ref, l_ref, sp_ref, wo_ref, bo_ref, g1_ref, b1_ref,
             w1_ref, bb1_ref, w2_ref, bb2_ref, g2_ref, b2_ref, o_ref):
        wv = wv_ref[...]
        z = wv[:, D:D + H]
        inv = 1.0 / (z + 1e-9)
        o2 = wv[:, :D] * _dot(inv, sp_ref[...])
        u = l_ref[...] + _dot(o2, wo_ref[...]) + bo_ref[...]
        u = _ln(u, g1_ref[...], b1_ref[...])
        f = u + _dot(jnp.maximum(_dot(u, w1_ref[...]) + bb1_ref[...], 0.0),
                     w2_ref[...]) + bb2_ref[...]
        o_ref[...] = _ln(f, g2_ref[...], b2_ref[...])

    row = lambda i: (i, 0)
    full = lambda i: (0, 0)
    return pl.pallas_call(
        body,
        grid=(E // BE,),
        in_specs=[
            pl.BlockSpec((BE, ROWW), row),
            pl.BlockSpec((BE, ED), row),
            pl.BlockSpec((H, D), full),
            pl.BlockSpec((D, ED), full),
            pl.BlockSpec((1, ED), full),
            pl.BlockSpec((1, ED), full),
            pl.BlockSpec((1, ED), full),
            pl.BlockSpec((ED, 4 * ED), full),
            pl.BlockSpec((1, 4 * ED), full),
            pl.BlockSpec((4 * ED, ED), full),
            pl.BlockSpec((1, ED), full),
            pl.BlockSpec((1, ED), full),
            pl.BlockSpec((1, ED), full),
        ],
        out_specs=pl.BlockSpec((BE, ED), row),
        out_shape=jax.ShapeDtypeStruct((E, ED), F32),
    )(wv2, lg, SPREAD, Wo, bo, g1, b1, W1, bb1, W2, bb2, g2, b2)


def _mesh():
    return plsc.VectorSubcoreMesh(core_axis_name="c", subcore_axis_name="s",
                                  num_cores=NC, num_subcores=NS)


_SC_PARAMS = pltpu.CompilerParams(needs_layout_passes=False,
                                  use_tc_tiling_on_sc=False)


def _gather_rows(tables, idxs, pairs, n_out, widths, G):
    per_w = n_out // (NC * NS)
    npairs = len(pairs)
    out_type = [jax.ShapeDtypeStruct((n_out, widths[t]), F32)
                for (t, _) in pairs]
    scratch = [pltpu.VMEM((G,), I32) for _ in idxs]
    scratch += [pltpu.VMEM((G, widths[t]), F32) for (t, _) in pairs]
    scratch += [pltpu.SemaphoreType.DMA, pltpu.SemaphoreType.DMA]

    @functools.partial(pl.kernel, mesh=_mesh(), compiler_params=_SC_PARAMS, out_type=tuple(out_type),
                       scratch_types=tuple(scratch))
    def k(*refs):
        nt = len(tables)
        ni = len(idxs)
        t_refs = refs[:nt]
        i_refs = refs[nt:nt + ni]
        o_refs = refs[nt + ni:nt + ni + npairs]
        ix_bufs = refs[nt + ni + npairs:nt + ni + npairs + ni]
        g_bufs = refs[nt + ni + npairs + ni:nt + ni + npairs + ni + npairs]
        sem_g, sem_w = refs[-2], refs[-1]

        wid = lax.axis_index("s") * NC + lax.axis_index("c")
        base = wid * per_w

        @pl.loop(0, per_w // G)
        def _(j):
            off = base + j * G
            for b in range(ni):
                pltpu.sync_copy(i_refs[b].at[pl.ds(off, G)], ix_bufs[b])
            descs = []
            for p, (t, b) in enumerate(pairs):
                descs.append(pltpu.async_copy(
                    t_refs[t].at[ix_bufs[b]], g_bufs[p], sem_g))
            for d in descs:
                d.wait()
            descs = []
            for p in range(npairs):
                descs.append(pltpu.async_copy(
                    g_bufs[p], o_refs[p].at[pl.ds(off, G)], sem_w))
            for d in descs:
                d.wait()

    return k(*tables, *idxs)


def _scat_dense(idx, wvals, n_edges, n_segs, G):
    per_core = n_edges // NC
    per_w = per_core // NS
    seg_share = -(-n_segs // (NS * 128)) * 128
    n_segs_pad = seg_share * NS

    @functools.partial(
        pl.kernel, mesh=_mesh(), compiler_params=_SC_PARAMS,
        out_type=jax.ShapeDtypeStruct((NC, n_segs_pad, ROWW), F32),
        scratch_types=(
            pltpu.VMEM((G,), I32),
            pltpu.VMEM((G, ROWW), F32),
            pltpu.VMEM((128, ROWW), F32),
            pltpu.VMEM_SHARED((n_segs_pad, ROWW), F32),
        ))
    def k(i_hbm, wv_hbm, o_hbm, ixb, buf, zbuf, spw):
        c = lax.axis_index("c")
        s = lax.axis_index("s")

        @pl.loop(0, 128)
        def _(r):
            for c0 in range(0, ROWW, LANES):
                zbuf[r, pl.ds(c0, LANES)] = jnp.zeros((LANES,), F32)

        lo = s * seg_share
        for t in range(seg_share // 128):
            pltpu.sync_copy(zbuf, spw.at[pl.ds(lo + t * 128, 128)])
        plsc.subcore_barrier()

        base = c * per_core + s * per_w

        @pl.loop(0, per_w // G)
        def _(j):
            off = base + j * G
            pltpu.sync_copy(i_hbm.at[pl.ds(off, G)], ixb)
            pltpu.sync_copy(wv_hbm.at[pl.ds(off, G)], buf)
            pltpu.sync_copy(buf, spw.at[ixb], add=True)

        plsc.subcore_barrier()
        pltpu.sync_copy(spw.at[pl.ds(lo, seg_share)],
                        o_hbm.at[c].at[pl.ds(lo, seg_share)])

    return k(idx, wvals)


def _scat_windowed(idx, wvals, n_edges, n_segs, wrows, wps):
    n_pad = NC * wps * wrows
    assert n_pad >= n_segs
    chunk = n_edges // NS
    CH = 2000
    assert chunk % CH == 0 and CH % LANES == 0
    nv = CH // LANES
    share = wrows // NS
    assert share % 8 == 0 and wrows % NS == 0

    @functools.partial(
        pl.kernel, mesh=_mesh(), compiler_params=_SC_PARAMS,
        out_type=jax.ShapeDtypeStruct((n_pad, ROWW), F32),
        scratch_types=(
            pltpu.VMEM((CH,), I32),
            pltpu.VMEM((2, 128), I32),
            pltpu.VMEM((2, 128), I32),
            pltpu.VMEM((128, ROWW), F32),
            pltpu.VMEM_SHARED((wrows + LANES, ROWW), F32),
        ))
    def k2(i_hbm, wv_hbm, o_hbm, ixb, ids2, loc2, gbuf, spw):
        c = lax.axis_index("c")
        s = lax.axis_index("s")
        iota = lax.iota(I32, LANES)
        trash = jnp.full((LANES,), wrows, I32) + iota

        def prefill(r):
            for c0 in range(0, 128, LANES):
                ids2[r, pl.ds(c0, LANES)] = iota
                loc2[r, pl.ds(c0, LANES)] = trash

        def flush(r):
            pltpu.sync_copy(wv_hbm.at[ids2.at[r]], gbuf)
            pltpu.sync_copy(gbuf, spw.at[loc2.at[r]], add=True)
            prefill(r)

        for w in range(wps):
            lo_seg = (c * wps + w) * wrows

            @pl.loop(0, 128)
            def _(r):
                for c0 in range(0, ROWW, LANES):
                    gbuf[r, pl.ds(c0, LANES)] = jnp.zeros((LANES,), F32)

            full, rem = divmod(share, 128)
            for t in range(full):
                pltpu.sync_copy(gbuf, spw.at[pl.ds(s * share + t * 128, 128)])
            if rem:
                pltpu.sync_copy(gbuf.at[pl.ds(0, rem)],
                                spw.at[pl.ds(s * share + full * 128, rem)])
            prefill(0)
            prefill(1)
            plsc.subcore_barrier()

            def chunk_body(t, fill):
                pltpu.sync_copy(i_hbm.at[pl.ds(s * chunk + t * CH, CH)], ixb)

                def body(i, fill):
                    d = ixb[pl.ds(i * LANES, LANES)]
                    m = (d >= lo_seg) & (d < lo_seg + wrows)
                    mi = m.astype(I32)
                    pos = fill + plsc.cumsum(mi) - 1
                    row = lax.shift_right_logical(pos, 7) & 1
                    col = pos & 127
                    eid = jnp.full((LANES,), s * chunk + t * CH + i * LANES,
                                   I32) + iota
                    plsc.store_scatter(ids2, [row, col], eid, mask=m)
                    plsc.store_scatter(loc2, [row, col], d - lo_seg, mask=m)
                    fill_new = fill + jnp.sum(mi)
                    crossed = lax.shift_right_logical(fill_new, 7) > \
                        lax.shift_right_logical(fill, 7)
                    par = lax.shift_right_logical(fill, 7) & 1

                    @pl.when(crossed & (par == 0))
                    def _():
                        flush(0)

                    @pl.when(crossed & (par == 1))
                    def _():
                        flush(1)

                    return fill_new

                return lax.fori_loop(0, nv, body, fill)

            fill = lax.fori_loop(0, chunk // CH, chunk_body, jnp.int32(0))
            rem_e = fill & 127
            par = lax.shift_right_logical(fill, 7) & 1

            @pl.when((rem_e > 0) & (par == 0))
            def _():
                flush(0)

            @pl.when((rem_e > 0) & (par == 1))
            def _():
                flush(1)

            plsc.subcore_barrier()
            pltpu.sync_copy(
                spw.at[pl.ds(s * share, share)],
                o_hbm.at[pl.ds(lo_seg + s * share, share)])
            plsc.subcore_barrier()

    return k2(idx, wvals)


def kernel(x, edge_feat, edge_index, lg_edge_index, rel_emb, params):
    N, D = x.shape
    E = edge_index.shape[1]
    ELG = lg_edge_index.shape[1]
    R, ED = rel_emb.shape
    L = params['nWq'].shape[0]

    BE = 6400
    BN = 2000
    G = 80

    WROWS = 12160
    WPS = -(-E // (NC * WROWS))

    src = edge_index[0]
    dst = edge_index[1]
    lsrc = lg_edge_index[0]
    ldst = lg_edge_index[1]

    hid = jnp.arange(D, dtype=I32) // DK
    SUM16 = (hid[:, None] == jnp.arange(H, dtype=I32)[None, :]).astype(F32)
    SPREAD = SUM16.T.reshape(H, D)

    feat3 = edge_feat.reshape(E // BE, 1, BE)
    lg = _lgx_call(feat3, rel_emb, E, R, ED, BE)

    zed = jnp.zeros((1, D), F32)
    for i in range(L):
        Wqkv = jnp.concatenate(
            [params['nWq'][i], params['nWk'][i], params['nWv'][i]], axis=1)
        bqkv = jnp.concatenate(
            [params['nbq'][i][None, :], zed, zed], axis=1)
        q, k, v = _qkv_call(x, Wqkv, bqkv, N, D, BN)

        qd, kg, vg, sxg, dxg = _gather_rows(
            (q, k, v, x), (src, dst),
            pairs=((0, 1), (1, 0), (2, 0), (3, 0), (3, 1)),
            n_out=E, widths=(D, D, D, D), G=G)

        wvals = _score_call(qd, kg, vg, lg, SUM16, SPREAD, E, D, ED, BE,
                            with_lg=True)
        part = _scat_dense(dst, wvals, E, N, G)
        nx = _npost_call(
            part[0], part[1], x, SPREAD,
            params['nWo'][i], params['nbo'][i][None, :],
            params['nlng'][i][None, :], params['nlnb'][i][None, :],
            params['nW1'][i], params['nb1'][i][None, :],
            params['nW2'][i], params['nb2'][i][None, :],
            params['nflng'][i][None, :], params['nflnb'][i][None, :],
            N, D, BN)

        We3 = jnp.concatenate(
            [params['eWq'][i], params['eWk'][i], params['eWv'][i]], axis=1)
        be3 = jnp.concatenate(
            [params['ebq'][i][None, :], zed, zed], axis=1)
        eq, ek, ev = _edense_call(lg, sxg, dxg, We3, be3, E, D, ED, BE)

        eql, ekl, evl = _gather_rows(
            (eq, ek, ev), (lsrc, ldst),
            pairs=((0, 1), (1, 0), (2, 0)),
            n_out=ELG, widths=(D, D, D), G=G)

        wvals2 = _score_call(eql, ekl, evl, None, SUM16, SPREAD, ELG, D, ED,
                             BE, with_lg=False)
        wv2 = _scat_windowed(ldst, wvals2, ELG, E, WROWS, WPS)
        lg = _epost_call(
            wv2, lg, SPREAD,
            params['eWo'][i], params['ebo'][i][None, :],
            params['elng'][i][None, :], params['elnb'][i][None, :],
            params['eW1'][i], params['eb1'][i][None, :],
            params['eW2'][i], params['eb2'][i][None, :],
            params['eflng'][i][None, :], params['eflnb'][i][None, :],
            E, D, ED, BE)
        x = nx

    return (x, lg)


# speedup vs baseline: 114.1003x; 3.4798x over previous
import functools
import math

import jax
import jax.numpy as jnp
from jax import lax
from jax.experimental import pallas as pl
from jax.experimental.pallas import tpu as pltpu
from jax.experimental.pallas import tpu_sc as plsc

F32 = jnp.float32
I32 = jnp.int32

NC = 2
NS = 16
LANES = 16

H = 8
DK = 16
ROWW = 144


def _ln(h, g, b):
    mu = jnp.mean(h, axis=-1, keepdims=True)
    d = h - mu
    var = jnp.mean(d * d, axis=-1, keepdims=True)
    return d * lax.rsqrt(var + 1e-5) * g + b


def _dot(a, b):
    return jnp.dot(a, b, preferred_element_type=F32)


def _lgx_call(feat3, rel_emb, E, R, ED, BF):
    def body(f_ref, t_ref, o_ref):
        f = f_ref[0, 0, :]
        oh = (f[:, None] == lax.broadcasted_iota(I32, (BF, R), 1)).astype(F32)
        o_ref[...] = _dot(oh, t_ref[...])

    return pl.pallas_call(
        body,
        grid=(E // BF,),
        in_specs=[
            pl.BlockSpec((1, 1, BF), lambda i: (i, 0, 0)),
            pl.BlockSpec((R, ED), lambda i: (0, 0)),
        ],
        out_specs=pl.BlockSpec((BF, ED), lambda i: (i, 0)),
        out_shape=jax.ShapeDtypeStruct((E, ED), F32),
    )(feat3, rel_emb)


def _qkv_call(x, W, b, N, D, BN):
    def body(x_ref, w_ref, b_ref, q_ref, k_ref, v_ref):
        o = _dot(x_ref[...], w_ref[...]) + b_ref[...]
        q_ref[...] = o[:, :D]
        k_ref[...] = o[:, D:2 * D]
        v_ref[...] = o[:, 2 * D:]

    outs = [jax.ShapeDtypeStruct((N, D), F32)] * 3
    return pl.pallas_call(
        body,
        grid=(N // BN,),
        in_specs=[
            pl.BlockSpec((BN, D), lambda i: (i, 0)),
            pl.BlockSpec((D, 3 * D), lambda i: (0, 0)),
            pl.BlockSpec((1, 3 * D), lambda i: (0, 0)),
        ],
        out_specs=[pl.BlockSpec((BN, D), lambda i: (i, 0))] * 3,
        out_shape=outs,
    )(x, W, b)


def _score_call(qd, kg, vg, lg, SUM16, SPREAD, M, D, ED, BE, with_lg):
    def body(*refs):
        if with_lg:
            q_ref, k_ref, v_ref, l_ref, s16_ref, sp_ref, o_ref = refs
            lgt = jnp.concatenate([l_ref[...]] * H, axis=1)
            ke = k_ref[...] + lgt
            ve = v_ref[...] + lgt
        else:
            q_ref, k_ref, v_ref, s16_ref, sp_ref, o_ref = refs
            ke = k_ref[...]
            ve = v_ref[...]
        prod = q_ref[...] * ke
        s = _dot(prod, s16_ref[...]) * (1.0 / 4.0)
        sc = jnp.exp(jnp.clip(s, -5.0, 5.0))
        wide = _dot(sc, sp_ref[...])
        wv = wide * ve
        o_ref[...] = jnp.concatenate(
            [wv, sc, jnp.zeros((BE, ROWW - D - H), F32)], axis=1)

    in_specs = [
        pl.BlockSpec((BE, D), lambda i: (i, 0)),
        pl.BlockSpec((BE, D), lambda i: (i, 0)),
        pl.BlockSpec((BE, D), lambda i: (i, 0)),
    ]
    args = [qd, kg, vg]
    if with_lg:
        in_specs.append(pl.BlockSpec((BE, ED), lambda i: (i, 0)))
        args.append(lg)
    in_specs += [
        pl.BlockSpec((D, H), lambda i: (0, 0)),
        pl.BlockSpec((H, D), lambda i: (0, 0)),
    ]
    args += [SUM16, SPREAD]
    return pl.pallas_call(
        body,
        grid=(M // BE,),
        in_specs=in_specs,
        out_specs=pl.BlockSpec((BE, ROWW), lambda i: (i, 0)),
        out_shape=jax.ShapeDtypeStruct((M, ROWW), F32),
    )(*args)


def _edense_call(lg, sxg, dxg, W3, b3, E, D, ED, BE):
    def body(l_ref, s_ref, d_ref, w_ref, b_ref, q_ref, k_ref, v_ref):
        o = _dot(l_ref[...], w_ref[...]) + b_ref[...]
        q_ref[...] = o[:, :D] + d_ref[...]
        k_ref[...] = o[:, D:2 * D] + s_ref[...]
        v_ref[...] = o[:, 2 * D:] + s_ref[...]

    outs = [jax.ShapeDtypeStruct((E, D), F32)] * 3
    return pl.pallas_call(
        body,
        grid=(E // BE,),
        in_specs=[
            pl.BlockSpec((BE, ED), lambda i: (i, 0)),
            pl.BlockSpec((BE, D), lambda i: (i, 0)),
            pl.BlockSpec((BE, D), lambda i: (i, 0)),
            pl.BlockSpec((ED, 3 * D), lambda i: (0, 0)),
            pl.BlockSpec((1, 3 * D), lambda i: (0, 0)),
        ],
        out_specs=[pl.BlockSpec((BE, D), lambda i: (i, 0))] * 3,
        out_shape=outs,
    )(lg, sxg, dxg, W3, b3)


def _npost_call(p0, p1, x, SPREAD, Wo, bo, g1, b1, W1, bb1, W2, bb2, g2, b2,
                N, D, BN):
    def body(p0_ref, p1_ref, x_ref, sp_ref, wo_ref, bo_ref, g1_ref, b1_ref,
             w1_ref, bb1_ref, w2_ref, bb2_ref, g2_ref, b2_ref, o_ref):
        wv = p0_ref[...] + p1_ref[...]
        z = wv[:, D:D + H]
        inv = 1.0 / (z + 1e-9)
        o = wv[:, :D] * _dot(inv, sp_ref[...])
        u = x_ref[...] + _dot(o, wo_ref[...]) + bo_ref[...]
        u = _ln(u, g1_ref[...], b1_ref[...])
        f = u + _dot(jnp.maximum(_dot(u, w1_ref[...]) + bb1_ref[...], 0.0),
                     w2_ref[...]) + bb2_ref[...]
        o_ref[...] = _ln(f, g2_ref[...], b2_ref[...])

    row = lambda i: (i, 0)
    full = lambda i: (0, 0)
    return pl.pallas_call(
        body,
        grid=(N // BN,),
        in_specs=[
            pl.BlockSpec((BN, ROWW), row),
            pl.BlockSpec((BN, ROWW), row),
            pl.BlockSpec((BN, D), row),
            pl.BlockSpec((H, D), full),
            pl.BlockSpec((D, D), full),
            pl.BlockSpec((1, D), full),
            pl.BlockSpec((1, D), full),
            pl.BlockSpec((1, D), full),
            pl.BlockSpec((D, 4 * D), full),
            pl.BlockSpec((1, 4 * D), full),
            pl.BlockSpec((4 * D, D), full),
            pl.BlockSpec((1, D), full),
            pl.BlockSpec((1, D), full),
            pl.BlockSpec((1, D), full),
        ],
        out_specs=pl.BlockSpec((BN, D), row),
        out_shape=jax.ShapeDtypeStruct((N, D), F32),
    )(p0, p1, x, SPREAD, Wo, bo, g1, b1, W1, bb1, W2, bb2, g2, b2)


def _epost_call(wv2, lg, SPREAD, Wo, bo, g1, b1, W1, bb1, W2, bb2, g2, b2,
                E, D, ED, BE):
    def body(wv_ref, l_ref, sp_ref, wo_ref, bo_ref, g1_ref, b1_ref,
             w1_ref, bb1_ref, w2_ref, bb2_ref, g2_ref, b2_ref, o_ref):
        wv = wv_ref[...]
        z = wv[:, D:D + H]
        inv = 1.0 / (z + 1e-9)
        o2 = wv[:, :D] * _dot(inv, sp_ref[...])
        u = l_ref[...] + _dot(o2, wo_ref[...]) + bo_ref[...]
        u = _ln(u, g1_ref[...], b1_ref[...])
        f = u + _dot(jnp.maximum(_dot(u, w1_ref[...]) + bb1_ref[...], 0.0),
                     w2_ref[...]) + bb2_ref[...]
        o_ref[...] = _ln(f, g2_ref[...], b2_ref[...])

    row = lambda i: (i, 0)
    full = lambda i: (0, 0)
    return pl.pallas_call(
        body,
        grid=(E // BE,),
        in_specs=[
            pl.BlockSpec((BE, ROWW), row),
            pl.BlockSpec((BE, ED), row),
            pl.BlockSpec((H, D), full),
            pl.BlockSpec((D, ED), full),
            pl.BlockSpec((1, ED), full),
            pl.BlockSpec((1, ED), full),
            pl.BlockSpec((1, ED), full),
            pl.BlockSpec((ED, 4 * ED), full),
            pl.BlockSpec((1, 4 * ED), full),
            pl.BlockSpec((4 * ED, ED), full),
            pl.BlockSpec((1, ED), full),
            pl.BlockSpec((1, ED), full),
            pl.BlockSpec((1, ED), full),
        ],
        out_specs=pl.BlockSpec((BE, ED), row),
        out_shape=jax.ShapeDtypeStruct((E, ED), F32),
    )(wv2, lg, SPREAD, Wo, bo, g1, b1, W1, bb1, W2, bb2, g2, b2)


def _mesh():
    return plsc.VectorSubcoreMesh(core_axis_name="c", subcore_axis_name="s",
                                  num_cores=NC, num_subcores=NS)


_SC_PARAMS = pltpu.CompilerParams(needs_layout_passes=False,
                                  use_tc_tiling_on_sc=False)


def _gather_rows(tables, idxs, pairs, n_out, widths, G):
    npairs = len(pairs)
    ni = len(idxs)
    nt = len(tables)
    out_type = [jax.ShapeDtypeStruct((n_out, widths[t]), F32)
                for (t, _) in pairs]

    @functools.partial(pl.kernel, mesh=_mesh(), compiler_params=_SC_PARAMS,
                       out_type=tuple(out_type),
                       scratch_types=(pltpu.SemaphoreType.DMA,))
    def k(*refs):
        t_refs = refs[:nt]
        i_refs = refs[nt:nt + ni]
        o_refs = refs[nt + ni:nt + ni + npairs]
        sem = refs[-1]

        def body(*bufs):
            ix_bufs = bufs[:ni]
            g_bufs = bufs[ni:]
            descs = []
            for p, (t, b) in enumerate(pairs):
                descs.append(pltpu.async_copy(
                    t_refs[t].at[ix_bufs[b].at[0]], g_bufs[p], sem))
            for d in descs:
                d.wait()

        pltpu.emit_pipeline(
            body,
            grid=(n_out // G,),
            in_specs=[pl.BlockSpec((1, G), lambda i: (0, i))
                      for _ in range(ni)],
            out_specs=[pl.BlockSpec((G, widths[t]), lambda i: (i, 0))
                       for (t, _) in pairs],
            core_axis_name=("c", "s"),
            dimension_semantics=(pltpu.PARALLEL,),
        )(*i_refs, *o_refs)

    return k(*tables, *[i.reshape(1, n_out) for i in idxs])


def _scat_dense(idx, wvals, n_edges, n_segs, G):
    seg_share = -(-n_segs // (NS * 128)) * 128
    n_segs_pad = seg_share * NS

    @functools.partial(
        pl.kernel, mesh=_mesh(), compiler_params=_SC_PARAMS,
        out_type=jax.ShapeDtypeStruct((NC, n_segs_pad, ROWW), F32),
        scratch_types=(
            pltpu.VMEM((64, ROWW), F32),
            pltpu.VMEM_SHARED((n_segs_pad, ROWW), F32),
        ))
    def k(i_hbm, wv_hbm, o_hbm, zbuf, spw):
        c = lax.axis_index("c")
        s = lax.axis_index("s")

        @pl.loop(0, 64)
        def _(r):
            for c0 in range(0, ROWW, LANES):
                zbuf[r, pl.ds(c0, LANES)] = jnp.zeros((LANES,), F32)

        lo = s * seg_share
        for t in range(seg_share // 64):
            pltpu.sync_copy(zbuf, spw.at[pl.ds(lo + t * 64, 64)])
        plsc.subcore_barrier()

        def body(ix_vmem, wv_vmem):
            pltpu.sync_copy(wv_vmem, spw.at[ix_vmem.at[0]], add=True)

        pltpu.emit_pipeline(
            body,
            grid=(n_edges // G,),
            in_specs=[
                pl.BlockSpec((1, G), lambda i: (0, i)),
                pl.BlockSpec((G, ROWW), lambda i: (i, 0)),
            ],
            core_axis_name=("c", "s"),
            dimension_semantics=(pltpu.PARALLEL,),
        )(i_hbm, wv_hbm)

        plsc.subcore_barrier()
        pltpu.sync_copy(spw.at[pl.ds(lo, seg_share)],
                        o_hbm.at[c].at[pl.ds(lo, seg_share)])

    return k(idx.reshape(1, n_edges), wvals)


def _scat_windowed(idx, wvals, n_edges, n_segs, wrows, wps):
    n_pad = NC * wps * wrows
    assert n_pad >= n_segs
    chunk = n_edges // NS
    CH = 2000
    assert chunk % CH == 0 and CH % LANES == 0
    nv = CH // LANES
    share = wrows // NS
    assert share % 8 == 0 and wrows % NS == 0

    @functools.partial(
        pl.kernel, mesh=_mesh(), compiler_params=_SC_PARAMS,
        out_type=jax.ShapeDtypeStruct((n_pad, ROWW), F32),
        scratch_types=(
            pltpu.VMEM((CH,), I32),
            pltpu.VMEM((2, 128), I32),
            pltpu.VMEM((2, 128), I32),
            pltpu.VMEM((128, ROWW), F32),
            pltpu.VMEM_SHARED((wrows + LANES, ROWW), F32),
        ))
    def k2(i_hbm, wv_hbm, o_hbm, ixb, ids2, loc2, gbuf, spw):
        c = lax.axis_index("c")
        s = lax.axis_index("s")
        iota = lax.iota(I32, LANES)
        trash = jnp.full((LANES,), wrows, I32) + iota

        def prefill(r):
            for c0 in range(0, 128, LANES):
                ids2[r, pl.ds(c0, LANES)] = iota
                loc2[r, pl.ds(c0, LANES)] = trash

        def flush(r):
            pltpu.sync_copy(wv_hbm.at[ids2.at[r]], gbuf)
            pltpu.sync_copy(gbuf, spw.at[loc2.at[r]], add=True)
            prefill(r)

        for w in range(wps):
            lo_seg = (c * wps + w) * wrows

            @pl.loop(0, 128)
            def _(r):
                for c0 in range(0, ROWW, LANES):
                    gbuf[r, pl.ds(c0, LANES)] = jnp.zeros((LANES,), F32)

            full, rem = divmod(share, 128)
            for t in range(full):
                pltpu.sync_copy(gbuf, spw.at[pl.ds(s * share + t * 128, 128)])
            if rem:
                pltpu.sync_copy(gbuf.at[pl.ds(0, rem)],
                                spw.at[pl.ds(s * share + full * 128, rem)])
            prefill(0)
            prefill(1)
            plsc.subcore_barrier()

            def chunk_body(t, fill):
                pltpu.sync_copy(i_hbm.at[pl.ds(s * chunk + t * CH, CH)], ixb)

                def body(i, fill):
                    d = ixb[pl.ds(i * LANES, LANES)]
                    m = (d >= lo_seg) & (d < lo_seg + wrows)
                    mi = m.astype(I32)
                    pos = fill + plsc.cumsum(mi) - 1
                    row = lax.shift_right_logical(pos, 7) & 1
                    col = pos & 127
                    eid = jnp.full((LANES,), s * chunk + t * CH + i * LANES,
                                   I32) + iota
                    plsc.store_scatter(ids2, [row, col], eid, mask=m)
                    plsc.store_scatter(loc2, [row, col], d - lo_seg, mask=m)
                    fill_new = fill + jnp.sum(mi)
                    crossed = lax.shift_right_logical(fill_new, 7) > \
                        lax.shift_right_logical(fill, 7)
                    par = lax.shift_right_logical(fill, 7) & 1

                    @pl.when(crossed & (par == 0))
                    def _():
                        flush(0)

                    @pl.when(crossed & (par == 1))
                    def _():
                        flush(1)

                    return fill_new

                return lax.fori_loop(0, nv, body, fill)

            fill = lax.fori_loop(0, chunk // CH, chunk_body, jnp.int32(0))
            rem_e = fill & 127
            par = lax.shift_right_logical(fill, 7) & 1

            @pl.when((rem_e > 0) & (par == 0))
            def _():
                flush(0)

            @pl.when((rem_e > 0) & (par == 1))
            def _():
                flush(1)

            plsc.subcore_barrier()
            pltpu.sync_copy(
                spw.at[pl.ds(s * share, share)],
                o_hbm.at[pl.ds(lo_seg + s * share, share)])
            plsc.subcore_barrier()

    return k2(idx, wvals)


def kernel(x, edge_feat, edge_index, lg_edge_index, rel_emb, params):
    N, D = x.shape
    E = edge_index.shape[1]
    ELG = lg_edge_index.shape[1]
    R, ED = rel_emb.shape
    L = params['nWq'].shape[0]

    BE = 6400
    BN = 2000
    G = 80

    WROWS = 12160
    WPS = -(-E // (NC * WROWS))

    src = edge_index[0]
    dst = edge_index[1]
    lsrc = lg_edge_index[0]
    ldst = lg_edge_index[1]

    hid = jnp.arange(D, dtype=I32) // DK
    SUM16 = (hid[:, None] == jnp.arange(H, dtype=I32)[None, :]).astype(F32)
    SPREAD = SUM16.T.reshape(H, D)

    feat3 = edge_feat.reshape(E // BE, 1, BE)
    lg = _lgx_call(feat3, rel_emb, E, R, ED, BE)

    zed = jnp.zeros((1, D), F32)
    for i in range(L):
        Wqkv = jnp.concatenate(
            [params['nWq'][i], params['nWk'][i], params['nWv'][i]], axis=1)
        bqkv = jnp.concatenate(
            [params['nbq'][i][None, :], zed, zed], axis=1)
        q, k, v = _qkv_call(x, Wqkv, bqkv, N, D, BN)

        qd, kg, vg, sxg, dxg = _gather_rows(
            (q, k, v, x), (src, dst),
            pairs=((0, 1), (1, 0), (2, 0), (3, 0), (3, 1)),
            n_out=E, widths=(D, D, D, D), G=G)

        wvals = _score_call(qd, kg, vg, lg, SUM16, SPREAD, E, D, ED, BE,
                            with_lg=True)
        part = _scat_dense(dst, wvals, E, N, G)
        nx = _npost_call(
            part[0], part[1], x, SPREAD,
            params['nWo'][i], params['nbo'][i][None, :],
            params['nlng'][i][None, :], params['nlnb'][i][None, :],
            params['nW1'][i], params['nb1'][i][None, :],
            params['nW2'][i], params['nb2'][i][None, :],
            params['nflng'][i][None, :], params['nflnb'][i][None, :],
            N, D, BN)

        We3 = jnp.concatenate(
            [params['eWq'][i], params['eWk'][i], params['eWv'][i]], axis=1)
        be3 = jnp.concatenate(
            [params['ebq'][i][None, :], zed, zed], axis=1)
        eq, ek, ev = _edense_call(lg, sxg, dxg, We3, be3, E, D, ED, BE)

        eql, ekl, evl = _gather_rows(
            (eq, ek, ev), (lsrc, ldst),
            pairs=((0, 1), (1, 0), (2, 0)),
            n_out=ELG, widths=(D, D, D), G=G)

        wvals2 = _score_call(eql, ekl, evl, None, SUM16, SPREAD, ELG, D, ED,
                             BE, with_lg=False)
        wv2 = _scat_windowed(ldst, wvals2, ELG, E, WROWS, WPS)
        lg = _epost_call(
            wv2, lg, SPREAD,
            params['eWo'][i], params['ebo'][i][None, :],
            params['elng'][i][None, :], params['elnb'][i][None, :],
            params['eW1'][i], params['eb1'][i][None, :],
            params['eW2'][i], params['eb2'][i][None, :],
            params['eflng'][i][None, :], params['eflnb'][i][None, :],
            E, D, ED, BE)
        x = nx

    return (x, lg)
